# baseline (device time: 13478 ns/iter reference)
import jax
import jax.numpy as jnp
from jax import lax
from jax.experimental import pallas as pl
from jax.experimental.pallas import tpu as pltpu

N_DEV = 8
N_CHUNK = 4


def kernel(x):
    m_per, n_per = x.shape
    rows = m_per // N_CHUNK

    def body(x_hbm, out_hbm, xv_ref, ov_ref, stats_ref,
             in_sems, out_sems, send_sems, recv_sems):
        me = lax.axis_index("i")

        barrier_sem = pltpu.get_barrier_semaphore()
        for d in range(1, N_DEV):
            pl.semaphore_signal(
                barrier_sem, inc=1,
                device_id=((me + d) % N_DEV,),
                device_id_type=pl.DeviceIdType.MESH,
            )

        copies_in = []
        for c in range(N_CHUNK):
            sl = pl.ds(c * rows, rows)
            cp = pltpu.make_async_copy(x_hbm.at[sl], xv_ref.at[sl], in_sems.at[c])
            cp.start()
            copies_in.append(cp)

        for c in range(N_CHUNK):
            sl = pl.ds(c * rows, rows)
            copies_in[c].wait()
            xc = xv_ref[sl, :]
            mc = jnp.max(xc, axis=1, keepdims=True)
            ec = jnp.exp(xc - mc)
            ov_ref[sl, :] = ec
            sc = jnp.sum(ec, axis=1, keepdims=True)
            cols = pl.ds(c * rows, rows)
            stats_ref[me, 0:1, cols] = mc.reshape(1, rows)
            stats_ref[me, 1:2, cols] = sc.reshape(1, rows)

        pl.semaphore_wait(barrier_sem, N_DEV - 1)

        sends = []
        for d in range(1, N_DEV):
            rdma = pltpu.make_async_remote_copy(
                src_ref=stats_ref.at[me],
                dst_ref=stats_ref.at[me],
                send_sem=send_sems.at[d - 1],
                recv_sem=recv_sems.at[me],
                device_id=((me + d) % N_DEV,),
                device_id_type=pl.DeviceIdType.MESH,
            )
            rdma.start()
            sends.append(rdma)

        for d in range(1, N_DEV):
            src = (me - d) % N_DEV
            recv = pltpu.make_async_remote_copy(
                src_ref=stats_ref.at[src],
                dst_ref=stats_ref.at[src],
                send_sem=send_sems.at[d - 1],
                recv_sem=recv_sems.at[src],
                device_id=(src,),
                device_id_type=pl.DeviceIdType.MESH,
            )
            recv.wait_recv()

        g = stats_ref[:, :, :]
        gm = g[:, 0:1, :]
        gs = g[:, 1:2, :]
        gmax = jnp.max(gm, axis=0)
        gsum = jnp.sum(gs * jnp.exp(gm - gmax[None]), axis=0)
        my_m = stats_ref[me, 0:1, :]
        scale = (jnp.exp(my_m - gmax) / gsum).reshape(m_per, 1)

        copies_out = []
        for c in range(N_CHUNK):
            sl = pl.ds(c * rows, rows)
            ov_ref[sl, :] = ov_ref[sl, :] * scale[c * rows:(c + 1) * rows, :]
            cp = pltpu.make_async_copy(ov_ref.at[sl], out_hbm.at[sl], out_sems.at[c])
            cp.start()
            copies_out.append(cp)

        for cp in copies_out:
            cp.wait()
        for rdma in sends:
            rdma.wait_send()

    return pl.pallas_call(
        body,
        out_shape=jax.ShapeDtypeStruct((m_per, n_per), jnp.float32),
        in_specs=[pl.BlockSpec(memory_space=pl.ANY)],
        out_specs=pl.BlockSpec(memory_space=pl.ANY),
        scratch_shapes=[
            pltpu.VMEM((m_per, n_per), jnp.float32),
            pltpu.VMEM((m_per, n_per), jnp.float32),
            pltpu.VMEM((N_DEV, 2, m_per), jnp.float32),
            pltpu.SemaphoreType.DMA((N_CHUNK,)),
            pltpu.SemaphoreType.DMA((N_CHUNK,)),
            pltpu.SemaphoreType.DMA((N_DEV - 1,)),
            pltpu.SemaphoreType.DMA((N_DEV,)),
        ],
        compiler_params=pltpu.CompilerParams(collective_id=0),
    )(x)


# device time: 11786 ns/iter; 1.1436x vs baseline; 1.1436x over previous
import jax
import jax.numpy as jnp
from jax import lax
from jax.experimental import pallas as pl
from jax.experimental.pallas import tpu as pltpu

N_DEV = 8


def kernel(x):
    m_per, n_per = x.shape

    def body(x_ref, out_ref, stats_ref, send_sems, recv_sems):
        me = lax.axis_index("i")

        barrier_sem = pltpu.get_barrier_semaphore()
        for d in range(1, N_DEV):
            pl.semaphore_signal(
                barrier_sem, inc=1,
                device_id=((me + d) % N_DEV,),
                device_id_type=pl.DeviceIdType.MESH,
            )

        eye = jnp.eye(m_per, dtype=jnp.float32)

        def col2row(v):
            return lax.dot_general(
                v, eye, (((0,), (0,)), ((), ())),
                preferred_element_type=jnp.float32)

        def row2col(v):
            return lax.dot_general(
                eye, v, (((1,), (1,)), ((), ())),
                preferred_element_type=jnp.float32)

        xv = x_ref[:, :]
        m = jnp.max(xv, axis=1, keepdims=True)
        e = jnp.exp(xv - m)
        out_ref[:, :] = e
        s = jnp.sum(e, axis=1, keepdims=True)

        stats_ref[me, 0:1, :] = col2row(m)
        stats_ref[me, 1:2, :] = col2row(s)

        pl.semaphore_wait(barrier_sem, N_DEV - 1)

        sends = []
        for d in range(1, N_DEV):
            rdma = pltpu.make_async_remote_copy(
                src_ref=stats_ref.at[me],
                dst_ref=stats_ref.at[me],
                send_sem=send_sems.at[d - 1],
                recv_sem=recv_sems.at[me],
                device_id=((me + d) % N_DEV,),
                device_id_type=pl.DeviceIdType.MESH,
            )
            rdma.start()
            sends.append(rdma)

        for d in range(1, N_DEV):
            src = (me - d) % N_DEV
            recv = pltpu.make_async_remote_copy(
                src_ref=stats_ref.at[src],
                dst_ref=stats_ref.at[src],
                send_sem=send_sems.at[d - 1],
                recv_sem=recv_sems.at[src],
                device_id=(src,),
                device_id_type=pl.DeviceIdType.MESH,
            )
            recv.wait_recv()

        g = stats_ref[:, :, :]
        gm = g[:, 0:1, :]
        gs = g[:, 1:2, :]
        gmax = jnp.max(gm, axis=0)
        gsum = jnp.sum(gs * jnp.exp(gm - gmax[None]), axis=0)
        my_m = stats_ref[me, 0:1, :]
        scale = row2col(jnp.exp(my_m - gmax) / gsum)
        out_ref[:, :] = out_ref[:, :] * scale

        for rdma in sends:
            rdma.wait_send()

    return pl.pallas_call(
        body,
        out_shape=jax.ShapeDtypeStruct((m_per, n_per), jnp.float32),
        in_specs=[pl.BlockSpec(memory_space=pltpu.VMEM)],
        out_specs=pl.BlockSpec(memory_space=pltpu.VMEM),
        scratch_shapes=[
            pltpu.VMEM((N_DEV, 2, m_per), jnp.float32),
            pltpu.SemaphoreType.DMA((N_DEV - 1,)),
            pltpu.SemaphoreType.DMA((N_DEV,)),
        ],
        compiler_params=pltpu.CompilerParams(collective_id=0),
    )(x)
